# baseline (speedup 1.0000x reference)
import jax
import jax.numpy as jnp
from jax.experimental import pallas as pl
from jax.experimental.pallas import tpu as pltpu
from jax.experimental.pallas import tpu_sc as plsc

BUCKET_SIZE = 64
NUM_HASH = 4
CHUNKS_PER_STEP = 8


def _hash_rank_body(xr_ref, qk_ref, buckets_ref, p_ref, aux_ref):
    b = pl.program_id(0)
    S = xr_ref.shape[1]
    X = xr_ref[0]
    ii = jax.lax.broadcasted_iota(jnp.int32, (S, S), 0)
    jj = jax.lax.broadcasted_iota(jnp.int32, (S, S), 1)
    tri = (jj < ii).astype(jnp.bfloat16)
    lane = jax.lax.broadcasted_iota(jnp.int32, (S, 128), 1)
    il = jax.lax.broadcasted_iota(jnp.int32, (S, 16), 1)
    ones_row = jnp.ones((8, S), jnp.bfloat16)
    for h in range(NUM_HASH):
        xh = X[:, 16 * h:16 * (h + 1)]
        mpos = jnp.max(xh, axis=1, keepdims=True)
        mneg = jnp.max(-xh, axis=1, keepdims=True)
        apos = jnp.min(jnp.where(xh == mpos, il, 64), axis=1, keepdims=True)
        aneg = jnp.min(jnp.where(-xh == mneg, il, 64), axis=1, keepdims=True)
        lb = jnp.where(mpos >= mneg, apos, 16 + aneg)
        oh = (lane == lb).astype(jnp.bfloat16)
        ranks = jax.lax.dot_general(
            tri, oh, (((1,), (0,)), ((), ())),
            preferred_element_type=jnp.float32)
        rank_i = jnp.sum(jnp.where(lane == lb, ranks, 0.0),
                         axis=1, keepdims=True)
        counts = jax.lax.dot_general(
            ones_row, oh, (((1,), (0,)), ((), ())),
            preferred_element_type=jnp.float32)
        counts0 = counts[0:1, :]
        start_i = jnp.sum(jnp.where(lane < lb, counts0, 0.0),
                          axis=1, keepdims=True)
        p_local = (start_i + rank_i).astype(jnp.int32)
        buckets_ref[0, h] = lb + 32 * h
        p_ref[0, h] = b * (NUM_HASH * S) + h * S + p_local
    qkr = qk_ref[0]
    sumsq = jnp.sum(qkr * qkr, axis=1, keepdims=True)
    rn = 1.0 / jnp.sqrt(sumsq)
    sidx = jax.lax.broadcasted_iota(jnp.int32, (S, 1), 0)
    sb = jax.lax.bitcast_convert_type(sidx, jnp.float32)
    aux_ref[0] = jnp.where(lane < 64, rn, sb)


def _attn_body(qs_ref, qb_ref, vs_ref, vb_ref, as_ref, ab_ref,
               qkv_ref, lse_ref):
    D = qs_ref.shape[2]
    CH = CHUNKS_PER_STEP
    BS = BUCKET_SIZE
    scale = float(D) ** -0.5
    for j in range(CH):
        q = qs_ref[0, j * BS:(j + 1) * BS]
        a_self = as_ref[0, j * BS:(j + 1) * BS]
        if j == 0:
            kprev = qb_ref[0]
            vprev = vb_ref[0]
            a_prev = ab_ref[0]
        else:
            kprev = qs_ref[0, (j - 1) * BS:j * BS]
            vprev = vs_ref[0, (j - 1) * BS:j * BS]
            a_prev = as_ref[0, (j - 1) * BS:j * BS]
        kcat = jnp.concatenate([q, kprev], axis=0)
        rn = jnp.concatenate([a_self[:, 0:1], a_prev[:, 0:1]], axis=0)
        kn = kcat * rn
        scores = jax.lax.dot_general(
            q, kn, (((1,), (1,)), ((), ())),
            preferred_element_type=jnp.float32) * scale
        qi = jax.lax.bitcast_convert_type(a_self[:, 64:65], jnp.int32)
        kvi = jnp.concatenate([a_self[:, 64:65], a_prev[:, 64:65]], axis=0)
        kvi_t = jax.lax.bitcast_convert_type(jnp.transpose(kvi), jnp.int32)
        m = (qi == kvi_t).astype(jnp.float32)
        scores = scores * (1.0 - m) - 100000.0 * m
        mx = jnp.max(scores, axis=1, keepdims=True)
        ex = jnp.exp(scores - mx)
        se = jnp.sum(ex, axis=1, keepdims=True)
        lse = mx + jnp.log(se)
        w = ex / se
        vcat = jnp.concatenate([vs_ref[0, j * BS:(j + 1) * BS], vprev],
                               axis=0)
        o = jax.lax.dot_general(
            w, vcat, (((1,), (0,)), ((), ())),
            preferred_element_type=jnp.float32)
        qkv_ref[0, j * BS:(j + 1) * BS] = o
        lse_ref[0, j * BS:(j + 1) * BS] = jnp.broadcast_to(lse, (BS, 128))


def _combine_body(qkv_ref, lse_ref, out_ref):
    l = lse_ref[0][:, :, 0:1]
    mx = jnp.max(l, axis=0, keepdims=True)
    r = jnp.exp(l - mx)
    s = jnp.sum(r, axis=0, keepdims=True)
    ratio = r / s
    out_ref[0] = jnp.sum(qkv_ref[0] * ratio, axis=0)


_VMESH = plsc.VectorSubcoreMesh(core_axis_name="c", subcore_axis_name="s")


def _sc_scatter_sorted(qk, v, aux, p):
    B, S, D = qk.shape
    H = p.shape[1]
    N = B * H * S
    gw = 32

    @pl.kernel(
        out_type=[
            jax.ShapeDtypeStruct((N, D), qk.dtype),
            jax.ShapeDtypeStruct((N, D), v.dtype),
            jax.ShapeDtypeStruct((N, 128), jnp.float32),
        ],
        mesh=_VMESH,
    )
    def sc_kernel(qk_hbm, v_hbm, aux_hbm, p_hbm, oq_hbm, ov_hbm, oa_hbm):
        def make_body(dest_hbm):
            def body(idxs, rows_vmem, p_vmem):
                _, _, w = idxs
                q = jax.lax.rem(w, 4)
                idx = p_vmem.at[0, 0, pl.ds(q * gw, gw)]
                pltpu.sync_copy(rows_vmem.at[0], dest_hbm.at[idx])
            return body

        def run(src_hbm, dest_hbm, width):
            pltpu.emit_pipeline(
                make_body(dest_hbm),
                grid=(B, H, S // gw),
                in_specs=[
                    pl.BlockSpec((1, gw, width), lambda b, h, w: (b, w, 0)),
                    pl.BlockSpec((1, 1, 128),
                                 lambda b, h, w: (b, h, w // 4)),
                ],
                out_specs=[],
                core_axis_name=("c", "s"),
                dimension_semantics=(pltpu.PARALLEL, pltpu.PARALLEL,
                                     pltpu.PARALLEL),
                _explicit_indices=True,
            )(src_hbm, p_hbm)

        run(qk_hbm, oq_hbm, D)
        run(v_hbm, ov_hbm, D)
        run(aux_hbm, oa_hbm, 128)

    return sc_kernel(qk, v, aux, p)


def _sc_gather_unsort(sorted_qkv, sorted_lse, p_flat):
    N, D = sorted_qkv.shape
    gw = 32

    @pl.kernel(
        out_type=[
            jax.ShapeDtypeStruct((N, D), sorted_qkv.dtype),
            jax.ShapeDtypeStruct((N, 128), sorted_lse.dtype),
        ],
        mesh=_VMESH,
    )
    def sc_kernel(rows_hbm, lse_hbm, p_hbm, orows_hbm, olse_hbm):
        def body(idxs, p_vmem, orows_vmem, olse_vmem):
            (w,) = idxs
            q = jax.lax.rem(w, 4)
            idx = p_vmem.at[0, pl.ds(q * gw, gw)]
            pltpu.sync_copy(rows_hbm.at[idx], orows_vmem)
            pltpu.sync_copy(lse_hbm.at[idx], olse_vmem)

        pltpu.emit_pipeline(
            body,
            grid=(N // gw,),
            in_specs=[pl.BlockSpec((1, 128), lambda w: (0, w // 4))],
            out_specs=[
                pl.BlockSpec((gw, D), lambda w: (w, 0)),
                pl.BlockSpec((gw, 128), lambda w: (w, 0)),
            ],
            core_axis_name=("c", "s"),
            dimension_semantics=(pltpu.PARALLEL,),
            _explicit_indices=True,
        )(p_hbm, orows_hbm, olse_hbm)

    return sc_kernel(sorted_qkv, sorted_lse, p_flat)


def kernel(qk, v, R):
    B, S, D = qk.shape
    H = NUM_HASH
    C = (S // BUCKET_SIZE) * H
    N = H * S
    CH = CHUNKS_PER_STEP
    W = C // CH

    xR = jnp.einsum('btf,bfhi->bthi', qk, R).reshape(B, S, H * 16)

    def run_batch_stage1(qk_b, v_b, xr_b):
        buckets4, p4, aux = pl.pallas_call(
            _hash_rank_body,
            grid=(1,),
            in_specs=[
                pl.BlockSpec((1, S, H * 16), lambda b: (b, 0, 0)),
                pl.BlockSpec((1, S, D), lambda b: (b, 0, 0)),
            ],
            out_specs=[
                pl.BlockSpec((1, H, S, 1), lambda b: (b, 0, 0, 0)),
                pl.BlockSpec((1, H, S, 1), lambda b: (b, 0, 0, 0)),
                pl.BlockSpec((1, S, 128), lambda b: (b, 0, 0)),
            ],
            out_shape=[
                jax.ShapeDtypeStruct((1, H, S, 1), jnp.int32),
                jax.ShapeDtypeStruct((1, H, S, 1), jnp.int32),
                jax.ShapeDtypeStruct((1, S, 128), jnp.float32),
            ],
        )(xr_b, qk_b)

        p3 = p4.reshape(1, H, S)
        sorted_qk, sorted_v, sorted_aux = _sc_scatter_sorted(
            qk_b, v_b, aux, p3)
        return (sorted_qk, sorted_v, sorted_aux, p3), buckets4.reshape(1, N)

    def run_batch_attn(sorted_qk, sorted_v, sorted_aux, p3):
        sqA = sorted_qk.reshape(W, CH * BUCKET_SIZE, D)
        svA = sorted_v.reshape(W, CH * BUCKET_SIZE, D)
        saA = sorted_aux.reshape(W, CH * BUCKET_SIZE, 128)
        sqB = sorted_qk.reshape(C, BUCKET_SIZE, D)
        svB = sorted_v.reshape(C, BUCKET_SIZE, D)
        saB = sorted_aux.reshape(C, BUCKET_SIZE, 128)

        def self_map(g):
            return (g, 0, 0)

        def bnd_map(g):
            return ((CH * g - 1) % C, 0, 0)

        sorted_qkv, sorted_lse = pl.pallas_call(
            _attn_body,
            grid=(W,),
            in_specs=[
                pl.BlockSpec((1, CH * BUCKET_SIZE, D), self_map),
                pl.BlockSpec((1, BUCKET_SIZE, D), bnd_map),
                pl.BlockSpec((1, CH * BUCKET_SIZE, D), self_map),
                pl.BlockSpec((1, BUCKET_SIZE, D), bnd_map),
                pl.BlockSpec((1, CH * BUCKET_SIZE, 128), self_map),
                pl.BlockSpec((1, BUCKET_SIZE, 128), bnd_map),
            ],
            out_specs=[
                pl.BlockSpec((1, CH * BUCKET_SIZE, D), self_map),
                pl.BlockSpec((1, CH * BUCKET_SIZE, 128), self_map),
            ],
            out_shape=[
                jax.ShapeDtypeStruct((W, CH * BUCKET_SIZE, D), jnp.float32),
                jax.ShapeDtypeStruct((W, CH * BUCKET_SIZE, 128),
                                     jnp.float32),
            ],
        )(sqA, sqB, svA, svB, saA, saB)
        return sorted_qkv, sorted_lse, p3

    def run_batch_gather(sorted_qkv, sorted_lse, p3):
        qkv_u, lse_u = _sc_gather_unsort(
            sorted_qkv.reshape(N, D),
            sorted_lse.reshape(N, 128),
            p3.reshape(1, N))
        return qkv_u.reshape(1, H, S, D), lse_u.reshape(1, H, S, 128)

    def run_batch_combine(qkv_u, lse_u):
        bs = 256
        return pl.pallas_call(
            _combine_body,
            grid=(1, S // bs),
            in_specs=[
                pl.BlockSpec((1, H, bs, D), lambda b, s: (b, 0, s, 0)),
                pl.BlockSpec((1, H, bs, 128), lambda b, s: (b, 0, s, 0)),
            ],
            out_specs=pl.BlockSpec((1, bs, D), lambda b, s: (b, s, 0)),
            out_shape=jax.ShapeDtypeStruct((1, S, D), jnp.float32),
        )(qkv_u, lse_u)

    def stage1(b):
        return run_batch_stage1(qk[b:b + 1], v[b:b + 1], xR[b:b + 1])

    st = [stage1(b) for b in range(B)]
    at = [run_batch_attn(*s[0]) for s in st]
    ga = [run_batch_gather(*a) for a in at]
    outs = [run_batch_combine(*g) for g in ga]
    out = jnp.concatenate(outs, axis=0)
    buckets = jnp.concatenate([s[1] for s in st], axis=0)
    return out, buckets


# speedup vs baseline: 1.0171x; 1.0171x over previous
import jax
import jax.numpy as jnp
from jax.experimental import pallas as pl
from jax.experimental.pallas import tpu as pltpu
from jax.experimental.pallas import tpu_sc as plsc

BUCKET_SIZE = 64
NUM_HASH = 4
CHUNKS_PER_STEP = 16


def _hash_rank_body(xr_ref, qk_ref, buckets_ref, p_ref, aux_ref):
    b = pl.program_id(0)
    S = xr_ref.shape[1]
    X = xr_ref[0]
    ii = jax.lax.broadcasted_iota(jnp.int32, (S, S), 0)
    jj = jax.lax.broadcasted_iota(jnp.int32, (S, S), 1)
    tri = (jj < ii).astype(jnp.bfloat16)
    lane = jax.lax.broadcasted_iota(jnp.int32, (S, 128), 1)
    il = jax.lax.broadcasted_iota(jnp.int32, (S, 16), 1)
    ones_row = jnp.ones((8, S), jnp.bfloat16)
    for h in range(NUM_HASH):
        xh = X[:, 16 * h:16 * (h + 1)]
        mpos = jnp.max(xh, axis=1, keepdims=True)
        mneg = jnp.max(-xh, axis=1, keepdims=True)
        apos = jnp.min(jnp.where(xh == mpos, il, 64), axis=1, keepdims=True)
        aneg = jnp.min(jnp.where(-xh == mneg, il, 64), axis=1, keepdims=True)
        lb = jnp.where(mpos >= mneg, apos, 16 + aneg)
        oh = (lane == lb).astype(jnp.bfloat16)
        ranks = jax.lax.dot_general(
            tri, oh, (((1,), (0,)), ((), ())),
            preferred_element_type=jnp.float32)
        rank_i = jnp.sum(jnp.where(lane == lb, ranks, 0.0),
                         axis=1, keepdims=True)
        counts = jax.lax.dot_general(
            ones_row, oh, (((1,), (0,)), ((), ())),
            preferred_element_type=jnp.float32)
        counts0 = counts[0:1, :]
        start_i = jnp.sum(jnp.where(lane < lb, counts0, 0.0),
                          axis=1, keepdims=True)
        p_local = (start_i + rank_i).astype(jnp.int32)
        buckets_ref[0, h] = lb + 32 * h
        p_ref[0, h] = b * (NUM_HASH * S) + h * S + p_local
    qkr = qk_ref[0]
    sumsq = jnp.sum(qkr * qkr, axis=1, keepdims=True)
    rn = 1.0 / jnp.sqrt(sumsq)
    sidx = jax.lax.broadcasted_iota(jnp.int32, (S, 1), 0)
    sb = jax.lax.bitcast_convert_type(sidx, jnp.float32)
    aux_ref[0] = jnp.where(lane < 64, rn, sb)


def _attn_body(qs_ref, qb_ref, vs_ref, vb_ref, as_ref, ab_ref,
               qkv_ref, lse_ref):
    D = qs_ref.shape[2]
    CH = CHUNKS_PER_STEP
    BS = BUCKET_SIZE
    scale = float(D) ** -0.5
    for j in range(CH):
        q = qs_ref[0, j * BS:(j + 1) * BS]
        a_self = as_ref[0, j * BS:(j + 1) * BS]
        if j == 0:
            kprev = qb_ref[0]
            vprev = vb_ref[0]
            a_prev = ab_ref[0]
        else:
            kprev = qs_ref[0, (j - 1) * BS:j * BS]
            vprev = vs_ref[0, (j - 1) * BS:j * BS]
            a_prev = as_ref[0, (j - 1) * BS:j * BS]
        kcat = jnp.concatenate([q, kprev], axis=0)
        rn = jnp.concatenate([a_self[:, 0:1], a_prev[:, 0:1]], axis=0)
        kn = kcat * rn
        scores = jax.lax.dot_general(
            q, kn, (((1,), (1,)), ((), ())),
            preferred_element_type=jnp.float32) * scale
        qi = jax.lax.bitcast_convert_type(a_self[:, 64:65], jnp.int32)
        kvi = jnp.concatenate([a_self[:, 64:65], a_prev[:, 64:65]], axis=0)
        kvi_t = jax.lax.bitcast_convert_type(jnp.transpose(kvi), jnp.int32)
        m = (qi == kvi_t).astype(jnp.float32)
        scores = scores * (1.0 - m) - 100000.0 * m
        mx = jnp.max(scores, axis=1, keepdims=True)
        ex = jnp.exp(scores - mx)
        se = jnp.sum(ex, axis=1, keepdims=True)
        lse = mx + jnp.log(se)
        w = ex / se
        vcat = jnp.concatenate([vs_ref[0, j * BS:(j + 1) * BS], vprev],
                               axis=0)
        o = jax.lax.dot_general(
            w, vcat, (((1,), (0,)), ((), ())),
            preferred_element_type=jnp.float32)
        qkv_ref[0, j * BS:(j + 1) * BS] = o
        lse_ref[0, j * BS:(j + 1) * BS] = jnp.broadcast_to(lse, (BS, 128))


def _combine_body(qkv_ref, lse_ref, out_ref):
    l = lse_ref[0][:, :, 0:1]
    mx = jnp.max(l, axis=0, keepdims=True)
    r = jnp.exp(l - mx)
    s = jnp.sum(r, axis=0, keepdims=True)
    ratio = r / s
    out_ref[0] = jnp.sum(qkv_ref[0] * ratio, axis=0)


_VMESH = plsc.VectorSubcoreMesh(core_axis_name="c", subcore_axis_name="s")


def _sc_scatter_sorted(qk, v, aux, p):
    B, S, D = qk.shape
    H = p.shape[1]
    N = B * H * S
    gw = 32

    @pl.kernel(
        out_type=[
            jax.ShapeDtypeStruct((N, D), qk.dtype),
            jax.ShapeDtypeStruct((N, D), v.dtype),
            jax.ShapeDtypeStruct((N, 128), jnp.float32),
        ],
        mesh=_VMESH,
    )
    def sc_kernel(qk_hbm, v_hbm, aux_hbm, p_hbm, oq_hbm, ov_hbm, oa_hbm):
        def make_body(dest_hbm):
            def body(idxs, rows_vmem, p_vmem):
                _, _, w = idxs
                q = jax.lax.rem(w, 4)
                idx = p_vmem.at[0, 0, pl.ds(q * gw, gw)]
                pltpu.sync_copy(rows_vmem.at[0], dest_hbm.at[idx])
            return body

        def run(src_hbm, dest_hbm, width):
            pltpu.emit_pipeline(
                make_body(dest_hbm),
                grid=(B, H, S // gw),
                in_specs=[
                    pl.BlockSpec((1, gw, width), lambda b, h, w: (b, w, 0)),
                    pl.BlockSpec((1, 1, 128),
                                 lambda b, h, w: (b, h, w // 4)),
                ],
                out_specs=[],
                core_axis_name=("c", "s"),
                dimension_semantics=(pltpu.PARALLEL, pltpu.PARALLEL,
                                     pltpu.PARALLEL),
                _explicit_indices=True,
            )(src_hbm, p_hbm)

        run(qk_hbm, oq_hbm, D)
        run(v_hbm, ov_hbm, D)
        run(aux_hbm, oa_hbm, 128)

    return sc_kernel(qk, v, aux, p)


def _sc_gather_unsort(sorted_qkv, sorted_lse, p_flat):
    N, D = sorted_qkv.shape
    gw = 32

    @pl.kernel(
        out_type=[
            jax.ShapeDtypeStruct((N, D), sorted_qkv.dtype),
            jax.ShapeDtypeStruct((N, 128), sorted_lse.dtype),
        ],
        mesh=_VMESH,
    )
    def sc_kernel(rows_hbm, lse_hbm, p_hbm, orows_hbm, olse_hbm):
        def body(idxs, p_vmem, orows_vmem, olse_vmem):
            (w,) = idxs
            q = jax.lax.rem(w, 4)
            idx = p_vmem.at[0, pl.ds(q * gw, gw)]
            pltpu.sync_copy(rows_hbm.at[idx], orows_vmem)
            pltpu.sync_copy(lse_hbm.at[idx], olse_vmem)

        pltpu.emit_pipeline(
            body,
            grid=(N // gw,),
            in_specs=[pl.BlockSpec((1, 128), lambda w: (0, w // 4))],
            out_specs=[
                pl.BlockSpec((gw, D), lambda w: (w, 0)),
                pl.BlockSpec((gw, 128), lambda w: (w, 0)),
            ],
            core_axis_name=("c", "s"),
            dimension_semantics=(pltpu.PARALLEL,),
            _explicit_indices=True,
        )(p_hbm, orows_hbm, olse_hbm)

    return sc_kernel(sorted_qkv, sorted_lse, p_flat)


def kernel(qk, v, R):
    B, S, D = qk.shape
    H = NUM_HASH
    C = (S // BUCKET_SIZE) * H
    N = H * S
    CH = CHUNKS_PER_STEP
    W = C // CH

    xR = jnp.einsum('btf,bfhi->bthi', qk, R).reshape(B, S, H * 16)

    def run_batch_stage1(qk_b, v_b, xr_b):
        buckets4, p4, aux = pl.pallas_call(
            _hash_rank_body,
            grid=(1,),
            in_specs=[
                pl.BlockSpec((1, S, H * 16), lambda b: (b, 0, 0)),
                pl.BlockSpec((1, S, D), lambda b: (b, 0, 0)),
            ],
            out_specs=[
                pl.BlockSpec((1, H, S, 1), lambda b: (b, 0, 0, 0)),
                pl.BlockSpec((1, H, S, 1), lambda b: (b, 0, 0, 0)),
                pl.BlockSpec((1, S, 128), lambda b: (b, 0, 0)),
            ],
            out_shape=[
                jax.ShapeDtypeStruct((1, H, S, 1), jnp.int32),
                jax.ShapeDtypeStruct((1, H, S, 1), jnp.int32),
                jax.ShapeDtypeStruct((1, S, 128), jnp.float32),
            ],
        )(xr_b, qk_b)

        p3 = p4.reshape(1, H, S)
        sorted_qk, sorted_v, sorted_aux = _sc_scatter_sorted(
            qk_b, v_b, aux, p3)
        return (sorted_qk, sorted_v, sorted_aux, p3), buckets4.reshape(1, N)

    def run_batch_attn(sorted_qk, sorted_v, sorted_aux, p3):
        sqA = sorted_qk.reshape(W, CH * BUCKET_SIZE, D)
        svA = sorted_v.reshape(W, CH * BUCKET_SIZE, D)
        saA = sorted_aux.reshape(W, CH * BUCKET_SIZE, 128)
        sqB = sorted_qk.reshape(C, BUCKET_SIZE, D)
        svB = sorted_v.reshape(C, BUCKET_SIZE, D)
        saB = sorted_aux.reshape(C, BUCKET_SIZE, 128)

        def self_map(g):
            return (g, 0, 0)

        def bnd_map(g):
            return ((CH * g - 1) % C, 0, 0)

        sorted_qkv, sorted_lse = pl.pallas_call(
            _attn_body,
            grid=(W,),
            in_specs=[
                pl.BlockSpec((1, CH * BUCKET_SIZE, D), self_map),
                pl.BlockSpec((1, BUCKET_SIZE, D), bnd_map),
                pl.BlockSpec((1, CH * BUCKET_SIZE, D), self_map),
                pl.BlockSpec((1, BUCKET_SIZE, D), bnd_map),
                pl.BlockSpec((1, CH * BUCKET_SIZE, 128), self_map),
                pl.BlockSpec((1, BUCKET_SIZE, 128), bnd_map),
            ],
            out_specs=[
                pl.BlockSpec((1, CH * BUCKET_SIZE, D), self_map),
                pl.BlockSpec((1, CH * BUCKET_SIZE, 128), self_map),
            ],
            out_shape=[
                jax.ShapeDtypeStruct((W, CH * BUCKET_SIZE, D), jnp.float32),
                jax.ShapeDtypeStruct((W, CH * BUCKET_SIZE, 128),
                                     jnp.float32),
            ],
        )(sqA, sqB, svA, svB, saA, saB)
        return sorted_qkv, sorted_lse, p3

    def run_batch_gather(sorted_qkv, sorted_lse, p3):
        qkv_u, lse_u = _sc_gather_unsort(
            sorted_qkv.reshape(N, D),
            sorted_lse.reshape(N, 128),
            p3.reshape(1, N))
        return qkv_u.reshape(1, H, S, D), lse_u.reshape(1, H, S, 128)

    def run_batch_combine(qkv_u, lse_u):
        bs = 256
        return pl.pallas_call(
            _combine_body,
            grid=(1, S // bs),
            in_specs=[
                pl.BlockSpec((1, H, bs, D), lambda b, s: (b, 0, s, 0)),
                pl.BlockSpec((1, H, bs, 128), lambda b, s: (b, 0, s, 0)),
            ],
            out_specs=pl.BlockSpec((1, bs, D), lambda b, s: (b, s, 0)),
            out_shape=jax.ShapeDtypeStruct((1, S, D), jnp.float32),
        )(qkv_u, lse_u)

    def stage1(b):
        return run_batch_stage1(qk[b:b + 1], v[b:b + 1], xR[b:b + 1])

    st = [stage1(b) for b in range(B)]
    at = [run_batch_attn(*s[0]) for s in st]
    ga = [run_batch_gather(*a) for a in at]
    outs = [run_batch_combine(*g) for g in ga]
    out = jnp.concatenate(outs, axis=0)
    buckets = jnp.concatenate([s[1] for s in st], axis=0)
    return out, buckets


# speedup vs baseline: 1.0222x; 1.0050x over previous
import jax
import jax.numpy as jnp
from jax.experimental import pallas as pl
from jax.experimental.pallas import tpu as pltpu
from jax.experimental.pallas import tpu_sc as plsc

BUCKET_SIZE = 64
NUM_HASH = 4
CHUNKS_PER_STEP = 32


def _hash_rank_body(xr_ref, qk_ref, buckets_ref, p_ref, aux_ref):
    b = pl.program_id(0)
    S = xr_ref.shape[1]
    X = xr_ref[0]
    ii = jax.lax.broadcasted_iota(jnp.int32, (S, S), 0)
    jj = jax.lax.broadcasted_iota(jnp.int32, (S, S), 1)
    tri = (jj < ii).astype(jnp.bfloat16)
    lane = jax.lax.broadcasted_iota(jnp.int32, (S, 128), 1)
    il = jax.lax.broadcasted_iota(jnp.int32, (S, 16), 1)
    ones_row = jnp.ones((8, S), jnp.bfloat16)
    for h in range(NUM_HASH):
        xh = X[:, 16 * h:16 * (h + 1)]
        mpos = jnp.max(xh, axis=1, keepdims=True)
        mneg = jnp.max(-xh, axis=1, keepdims=True)
        apos = jnp.min(jnp.where(xh == mpos, il, 64), axis=1, keepdims=True)
        aneg = jnp.min(jnp.where(-xh == mneg, il, 64), axis=1, keepdims=True)
        lb = jnp.where(mpos >= mneg, apos, 16 + aneg)
        oh = (lane == lb).astype(jnp.bfloat16)
        ranks = jax.lax.dot_general(
            tri, oh, (((1,), (0,)), ((), ())),
            preferred_element_type=jnp.float32)
        rank_i = jnp.sum(jnp.where(lane == lb, ranks, 0.0),
                         axis=1, keepdims=True)
        counts = jax.lax.dot_general(
            ones_row, oh, (((1,), (0,)), ((), ())),
            preferred_element_type=jnp.float32)
        counts0 = counts[0:1, :]
        start_i = jnp.sum(jnp.where(lane < lb, counts0, 0.0),
                          axis=1, keepdims=True)
        p_local = (start_i + rank_i).astype(jnp.int32)
        buckets_ref[0, h] = lb + 32 * h
        p_ref[0, h] = b * (NUM_HASH * S) + h * S + p_local
    qkr = qk_ref[0]
    sumsq = jnp.sum(qkr * qkr, axis=1, keepdims=True)
    rn = 1.0 / jnp.sqrt(sumsq)
    sidx = jax.lax.broadcasted_iota(jnp.int32, (S, 1), 0)
    sb = jax.lax.bitcast_convert_type(sidx, jnp.float32)
    aux_ref[0] = jnp.where(lane < 64, rn, sb)


def _attn_body(qs_ref, qb_ref, vs_ref, vb_ref, as_ref, ab_ref,
               qkv_ref, lse_ref):
    D = qs_ref.shape[2]
    CH = CHUNKS_PER_STEP
    BS = BUCKET_SIZE
    scale = float(D) ** -0.5
    for j in range(CH):
        q = qs_ref[0, j * BS:(j + 1) * BS]
        a_self = as_ref[0, j * BS:(j + 1) * BS]
        if j == 0:
            kprev = qb_ref[0]
            vprev = vb_ref[0]
            a_prev = ab_ref[0]
        else:
            kprev = qs_ref[0, (j - 1) * BS:j * BS]
            vprev = vs_ref[0, (j - 1) * BS:j * BS]
            a_prev = as_ref[0, (j - 1) * BS:j * BS]
        kcat = jnp.concatenate([q, kprev], axis=0)
        rn = jnp.concatenate([a_self[:, 0:1], a_prev[:, 0:1]], axis=0)
        kn = kcat * rn
        scores = jax.lax.dot_general(
            q, kn, (((1,), (1,)), ((), ())),
            preferred_element_type=jnp.float32) * scale
        qi = jax.lax.bitcast_convert_type(a_self[:, 64:65], jnp.int32)
        kvi = jnp.concatenate([a_self[:, 64:65], a_prev[:, 64:65]], axis=0)
        kvi_t = jax.lax.bitcast_convert_type(jnp.transpose(kvi), jnp.int32)
        m = (qi == kvi_t).astype(jnp.float32)
        scores = scores * (1.0 - m) - 100000.0 * m
        mx = jnp.max(scores, axis=1, keepdims=True)
        ex = jnp.exp(scores - mx)
        se = jnp.sum(ex, axis=1, keepdims=True)
        lse = mx + jnp.log(se)
        w = ex / se
        vcat = jnp.concatenate([vs_ref[0, j * BS:(j + 1) * BS], vprev],
                               axis=0)
        o = jax.lax.dot_general(
            w, vcat, (((1,), (0,)), ((), ())),
            preferred_element_type=jnp.float32)
        qkv_ref[0, j * BS:(j + 1) * BS] = o
        lse_ref[0, j * BS:(j + 1) * BS] = jnp.broadcast_to(lse, (BS, 128))


def _combine_body(qkv_ref, lse_ref, out_ref):
    l = lse_ref[0][:, :, 0:1]
    mx = jnp.max(l, axis=0, keepdims=True)
    r = jnp.exp(l - mx)
    s = jnp.sum(r, axis=0, keepdims=True)
    ratio = r / s
    out_ref[0] = jnp.sum(qkv_ref[0] * ratio, axis=0)


_VMESH = plsc.VectorSubcoreMesh(core_axis_name="c", subcore_axis_name="s")


def _sc_scatter_sorted(qk, v, aux, p):
    B, S, D = qk.shape
    H = p.shape[1]
    N = B * H * S
    gw = 32

    @pl.kernel(
        out_type=[
            jax.ShapeDtypeStruct((N, D), qk.dtype),
            jax.ShapeDtypeStruct((N, D), v.dtype),
            jax.ShapeDtypeStruct((N, 128), jnp.float32),
        ],
        mesh=_VMESH,
    )
    def sc_kernel(qk_hbm, v_hbm, aux_hbm, p_hbm, oq_hbm, ov_hbm, oa_hbm):
        def make_body(dest_hbm):
            def body(idxs, rows_vmem, p_vmem):
                _, _, w = idxs
                q = jax.lax.rem(w, 4)
                idx = p_vmem.at[0, 0, pl.ds(q * gw, gw)]
                pltpu.sync_copy(rows_vmem.at[0], dest_hbm.at[idx])
            return body

        def run(src_hbm, dest_hbm, width):
            pltpu.emit_pipeline(
                make_body(dest_hbm),
                grid=(B, H, S // gw),
                in_specs=[
                    pl.BlockSpec((1, gw, width), lambda b, h, w: (b, w, 0)),
                    pl.BlockSpec((1, 1, 128),
                                 lambda b, h, w: (b, h, w // 4)),
                ],
                out_specs=[],
                core_axis_name=("c", "s"),
                dimension_semantics=(pltpu.PARALLEL, pltpu.PARALLEL,
                                     pltpu.PARALLEL),
                _explicit_indices=True,
            )(src_hbm, p_hbm)

        run(qk_hbm, oq_hbm, D)
        run(v_hbm, ov_hbm, D)
        run(aux_hbm, oa_hbm, 128)

    return sc_kernel(qk, v, aux, p)


def _sc_gather_unsort(sorted_qkv, sorted_lse, p_flat):
    N, D = sorted_qkv.shape
    gw = 32

    @pl.kernel(
        out_type=[
            jax.ShapeDtypeStruct((N, D), sorted_qkv.dtype),
            jax.ShapeDtypeStruct((N, 128), sorted_lse.dtype),
        ],
        mesh=_VMESH,
    )
    def sc_kernel(rows_hbm, lse_hbm, p_hbm, orows_hbm, olse_hbm):
        def body(idxs, p_vmem, orows_vmem, olse_vmem):
            (w,) = idxs
            q = jax.lax.rem(w, 4)
            idx = p_vmem.at[0, pl.ds(q * gw, gw)]
            pltpu.sync_copy(rows_hbm.at[idx], orows_vmem)
            pltpu.sync_copy(lse_hbm.at[idx], olse_vmem)

        pltpu.emit_pipeline(
            body,
            grid=(N // gw,),
            in_specs=[pl.BlockSpec((1, 128), lambda w: (0, w // 4))],
            out_specs=[
                pl.BlockSpec((gw, D), lambda w: (w, 0)),
                pl.BlockSpec((gw, 128), lambda w: (w, 0)),
            ],
            core_axis_name=("c", "s"),
            dimension_semantics=(pltpu.PARALLEL,),
            _explicit_indices=True,
        )(p_hbm, orows_hbm, olse_hbm)

    return sc_kernel(sorted_qkv, sorted_lse, p_flat)


def kernel(qk, v, R):
    B, S, D = qk.shape
    H = NUM_HASH
    C = (S // BUCKET_SIZE) * H
    N = H * S
    CH = CHUNKS_PER_STEP
    W = C // CH

    xR = jnp.einsum('btf,bfhi->bthi', qk, R).reshape(B, S, H * 16)

    def run_batch_stage1(qk_b, v_b, xr_b):
        buckets4, p4, aux = pl.pallas_call(
            _hash_rank_body,
            grid=(1,),
            in_specs=[
                pl.BlockSpec((1, S, H * 16), lambda b: (b, 0, 0)),
                pl.BlockSpec((1, S, D), lambda b: (b, 0, 0)),
            ],
            out_specs=[
                pl.BlockSpec((1, H, S, 1), lambda b: (b, 0, 0, 0)),
                pl.BlockSpec((1, H, S, 1), lambda b: (b, 0, 0, 0)),
                pl.BlockSpec((1, S, 128), lambda b: (b, 0, 0)),
            ],
            out_shape=[
                jax.ShapeDtypeStruct((1, H, S, 1), jnp.int32),
                jax.ShapeDtypeStruct((1, H, S, 1), jnp.int32),
                jax.ShapeDtypeStruct((1, S, 128), jnp.float32),
            ],
        )(xr_b, qk_b)

        p3 = p4.reshape(1, H, S)
        sorted_qk, sorted_v, sorted_aux = _sc_scatter_sorted(
            qk_b, v_b, aux, p3)
        return (sorted_qk, sorted_v, sorted_aux, p3), buckets4.reshape(1, N)

    def run_batch_attn(sorted_qk, sorted_v, sorted_aux, p3):
        sqA = sorted_qk.reshape(W, CH * BUCKET_SIZE, D)
        svA = sorted_v.reshape(W, CH * BUCKET_SIZE, D)
        saA = sorted_aux.reshape(W, CH * BUCKET_SIZE, 128)
        sqB = sorted_qk.reshape(C, BUCKET_SIZE, D)
        svB = sorted_v.reshape(C, BUCKET_SIZE, D)
        saB = sorted_aux.reshape(C, BUCKET_SIZE, 128)

        def self_map(g):
            return (g, 0, 0)

        def bnd_map(g):
            return ((CH * g - 1) % C, 0, 0)

        sorted_qkv, sorted_lse = pl.pallas_call(
            _attn_body,
            grid=(W,),
            in_specs=[
                pl.BlockSpec((1, CH * BUCKET_SIZE, D), self_map),
                pl.BlockSpec((1, BUCKET_SIZE, D), bnd_map),
                pl.BlockSpec((1, CH * BUCKET_SIZE, D), self_map),
                pl.BlockSpec((1, BUCKET_SIZE, D), bnd_map),
                pl.BlockSpec((1, CH * BUCKET_SIZE, 128), self_map),
                pl.BlockSpec((1, BUCKET_SIZE, 128), bnd_map),
            ],
            out_specs=[
                pl.BlockSpec((1, CH * BUCKET_SIZE, D), self_map),
                pl.BlockSpec((1, CH * BUCKET_SIZE, 128), self_map),
            ],
            out_shape=[
                jax.ShapeDtypeStruct((W, CH * BUCKET_SIZE, D), jnp.float32),
                jax.ShapeDtypeStruct((W, CH * BUCKET_SIZE, 128),
                                     jnp.float32),
            ],
        )(sqA, sqB, svA, svB, saA, saB)
        return sorted_qkv, sorted_lse, p3

    def run_batch_gather(sorted_qkv, sorted_lse, p3):
        qkv_u, lse_u = _sc_gather_unsort(
            sorted_qkv.reshape(N, D),
            sorted_lse.reshape(N, 128),
            p3.reshape(1, N))
        return qkv_u.reshape(1, H, S, D), lse_u.reshape(1, H, S, 128)

    def run_batch_combine(qkv_u, lse_u):
        bs = 512
        return pl.pallas_call(
            _combine_body,
            grid=(1, S // bs),
            in_specs=[
                pl.BlockSpec((1, H, bs, D), lambda b, s: (b, 0, s, 0)),
                pl.BlockSpec((1, H, bs, 128), lambda b, s: (b, 0, s, 0)),
            ],
            out_specs=pl.BlockSpec((1, bs, D), lambda b, s: (b, s, 0)),
            out_shape=jax.ShapeDtypeStruct((1, S, D), jnp.float32),
        )(qkv_u, lse_u)

    def stage1(b):
        return run_batch_stage1(qk[b:b + 1], v[b:b + 1], xR[b:b + 1])

    st = [stage1(b) for b in range(B)]
    at = [run_batch_attn(*s[0]) for s in st]
    ga = [run_batch_gather(*a) for a in at]
    outs = [run_batch_combine(*g) for g in ga]
    out = jnp.concatenate(outs, axis=0)
    buckets = jnp.concatenate([s[1] for s in st], axis=0)
    return out, buckets
